# baseline (speedup 1.0000x reference)
import functools
import jax
import jax.numpy as jnp
from jax import lax
from jax.experimental import pallas as pl
from jax.experimental.pallas import tpu as pltpu
from jax.experimental.pallas import tpu_sc as plsc

N = 10000
E = 320000
F_IN = 128
F = 256
H = 128
Q = 2
W64 = H // Q
NC, NS = 2, 16
C = 80
NBUF = 5
EPT = E // NS
NCH = EPT // C
EPW = E // (NC * NS)
NCH_D = EPW // C
RPT = 624
RB = 1000

_mesh = plsc.VectorSubcoreMesh(core_axis_name="c", subcore_axis_name="s",
                               num_cores=NC, num_subcores=NS)


def _zero_vmem(buf, rows, width):
    z = jnp.zeros((16,), jnp.float32)

    def zr(r, _):
        def zc(k, _):
            buf[r, pl.ds(k * 16, 16)] = z
            return 0
        lax.fori_loop(0, width // 16, zc, 0)
        return 0
    lax.fori_loop(0, rows, zr, 0)


def _zero_stripe(acc, zbuf, base, rows, chunk):
    nfull = rows // chunk
    rem = rows % chunk
    for t in range(nfull):
        pltpu.sync_copy(zbuf, acc.at[pl.ds(base + t * chunk, chunk)])
    if rem:
        pltpu.sync_copy(zbuf.at[pl.ds(0, rem)],
                        acc.at[pl.ds(base + nfull * chunk, rem)])


@functools.partial(
    pl.kernel,
    out_type=jax.ShapeDtypeStruct((NC, N, 16), jnp.float32),
    mesh=_mesh,
    scratch_types=[
        pltpu.VMEM((NCH_D, C), jnp.int32),
        pltpu.VMEM((C, 16), jnp.float32),
        pltpu.VMEM((C, 16), jnp.float32),
        pltpu.VMEM_SHARED((N, 16), jnp.float32),
    ],
    compiler_params=pltpu.CompilerParams(use_tc_tiling_on_sc=False, skip_device_barrier=True),
)
def _deg_kernel(dst_hbm, out_hbm, dstbuf, onesbuf, zbuf, acc):
    c = lax.axis_index("c")
    s = lax.axis_index("s")
    w = s * NC + c
    e0 = jnp.where(lax.iota(jnp.int32, 16) == 0, 1.0, 0.0).astype(jnp.float32)
    z = jnp.zeros((16,), jnp.float32)

    def fill(r, _):
        onesbuf[r] = e0
        zbuf[r] = z
        return 0
    lax.fori_loop(0, C, fill, 0)

    base = pl.multiple_of(s * RPT, 8)
    _zero_stripe(acc, zbuf, base, RPT, C)

    @pl.when(s == NS - 1)
    def _():
        _zero_stripe(acc, zbuf, N - 16, 16, C)

    plsc.subcore_barrier()

    pltpu.sync_copy(dst_hbm.at[w], dstbuf)

    def body(j, _):
        pltpu.sync_copy(onesbuf, acc.at[dstbuf.at[j]], add=True)
        return 0
    lax.fori_loop(0, NCH_D, body, 0)

    plsc.subcore_barrier()
    pltpu.sync_copy(acc.at[pl.ds(base, RPT)],
                    out_hbm.at[c].at[pl.ds(base, RPT)])

    @pl.when(s == NS - 1)
    def _():
        pltpu.sync_copy(acc.at[pl.ds(N - 16, 16)],
                        out_hbm.at[c].at[pl.ds(N - 16, 16)])


@functools.partial(
    pl.kernel,
    out_type=jax.ShapeDtypeStruct((NC, Q, N, W64), jnp.float32),
    mesh=_mesh,
    scratch_types=[
        pltpu.VMEM((NCH, C), jnp.int32),
        pltpu.VMEM((NCH, C), jnp.int32),
        [pltpu.VMEM((C, W64), jnp.float32) for _ in range(NBUF)],
        pltpu.VMEM((C, W64), jnp.float32),
        pltpu.VMEM_SHARED((N, W64), jnp.float32),
        [pltpu.SemaphoreType.DMA for _ in range(NBUF)],
        [pltpu.SemaphoreType.DMA for _ in range(NBUF)],
    ],
    compiler_params=pltpu.CompilerParams(use_tc_tiling_on_sc=False, skip_device_barrier=True),
)
def _prop_kernel(h4_hbm, src_hbm, dst_hbm, out_hbm,
                 srcbuf, dstbuf, gbufs, zbuf, acc, gsems, ssems):
    c = lax.axis_index("c")
    s = lax.axis_index("s")

    _zero_vmem(zbuf, C, W64)
    base = pl.multiple_of(s * RPT, 8)

    pltpu.sync_copy(src_hbm.at[s], srcbuf)
    pltpu.sync_copy(dst_hbm.at[s], dstbuf)

    for q in range(Q):
        _zero_stripe(acc, zbuf, base, RPT, C)

        @pl.when(s == NS - 1)
        def _():
            _zero_stripe(acc, zbuf, N - 16, 16, C)

        plsc.subcore_barrier()

        tab = h4_hbm.at[c].at[q]

        for b in range(NBUF):
            pltpu.async_copy(tab.at[srcbuf.at[b]], gbufs[b], gsems[b])

        def body(i, _):
            sds = []
            for b in range(NBUF):
                j = i * NBUF + b
                pltpu.make_async_copy(tab.at[srcbuf.at[0]],
                                      gbufs[b], gsems[b]).wait()
                sds.append(pltpu.async_copy(gbufs[b], acc.at[dstbuf.at[j]],
                                            ssems[b], add=True))
            for b in range(NBUF):
                j = i * NBUF + b + NBUF
                sds[b].wait()

                @pl.when(j < NCH)
                def _():
                    pltpu.async_copy(tab.at[srcbuf.at[j]], gbufs[b], gsems[b])
            return 0
        lax.fori_loop(0, NCH // NBUF, body, 0)

        plsc.subcore_barrier()

        outq = out_hbm.at[c].at[q]
        pltpu.sync_copy(acc.at[pl.ds(base, RPT)], outq.at[pl.ds(base, RPT)])

        @pl.when(s == NS - 1)
        def _():
            pltpu.sync_copy(acc.at[pl.ds(N - 16, 16)],
                            outq.at[pl.ds(N - 16, 16)])

        plsc.subcore_barrier()


def _dis_from_degp(degp_blk):
    deg = jnp.sum(degp_blk, axis=(0, 2)) + 1.0
    return lax.rsqrt(deg)


def _hp0_body(x_ref, degp_ref, o_ref):
    c = pl.program_id(0)
    deg = jnp.sum(degp_ref[...], axis=(0, 2)) + 1.0
    rdis = jnp.sqrt(deg).reshape(RB, 1)
    v = jnp.where(c == 0, x_ref[...] * rdis, 0.0)
    for qi in range(Q):
        o_ref[0, qi] = v[:, qi * W64:(qi + 1) * W64]


def _mm_body(s_ref, hp_ref, degp_ref, w_ref, flag_ref, o_ref):
    dis = _dis_from_degp(degp_ref[...]).reshape(1, 1, RB, 1)
    v = s_ref[...] + hp_ref[...]
    is_first = flag_ref[0, 0] > 0.0
    act = dis * jnp.where(is_first, v, jax.nn.relu(v))
    w = w_ref[0, 0]
    h = jnp.zeros((RB, W64), jnp.float32)
    for ci in range(NC):
        for qi in range(Q):
            wslice = w[:, ci * H + qi * W64: ci * H + (qi + 1) * W64]
            h = h + lax.dot_general(act[ci, qi], wslice,
                                    (((1,), (1,)), ((), ())),
                                    preferred_element_type=jnp.float32)
    o_ref[...] = (dis[0, 0] * h)[None, None]


def _final_body(s_ref, hp_ref, degp_ref, o_ref):
    dis = _dis_from_degp(degp_ref[...]).reshape(1, 1, RB, 1)
    v = dis * (s_ref[...] + hp_ref[...])
    for ci in range(NC):
        for qi in range(Q):
            lo = ci * H + qi * W64
            o_ref[:, lo:lo + W64] = v[ci, qi]


def _hp0(x, degp):
    return pl.pallas_call(
        _hp0_body,
        out_shape=jax.ShapeDtypeStruct((NC, Q, N, W64), jnp.float32),
        grid=(NC, N // RB),
        in_specs=[
            pl.BlockSpec((RB, F_IN), lambda c, r: (r, 0)),
            pl.BlockSpec((NC, RB, 16), lambda c, r: (0, r, 0)),
        ],
        out_specs=pl.BlockSpec((1, Q, RB, W64), lambda c, r: (c, 0, r, 0)),
    )(x, degp)


def _mm(s, hp, degp, wr, flag):
    return pl.pallas_call(
        _mm_body,
        out_shape=jax.ShapeDtypeStruct((NC, Q, N, W64), jnp.float32),
        grid=(NC, Q, N // RB),
        in_specs=[
            pl.BlockSpec((NC, Q, RB, W64), lambda c, q, r: (0, 0, r, 0)),
            pl.BlockSpec((NC, Q, RB, W64), lambda c, q, r: (0, 0, r, 0)),
            pl.BlockSpec((NC, RB, 16), lambda c, q, r: (0, r, 0)),
            pl.BlockSpec((1, 1, W64, F), lambda c, q, r: (c, q, 0, 0)),
            pl.BlockSpec((1, 1), lambda c, q, r: (0, 0)),
        ],
        out_specs=pl.BlockSpec((1, 1, RB, W64), lambda c, q, r: (c, q, r, 0)),
    )(s, hp, degp, wr, flag)


def _final(s, hp, degp):
    return pl.pallas_call(
        _final_body,
        out_shape=jax.ShapeDtypeStruct((N, F), jnp.float32),
        grid=(N // RB,),
        in_specs=[
            pl.BlockSpec((NC, Q, RB, W64), lambda r: (0, 0, r, 0)),
            pl.BlockSpec((NC, Q, RB, W64), lambda r: (0, 0, r, 0)),
            pl.BlockSpec((NC, RB, 16), lambda r: (0, r, 0)),
        ],
        out_specs=pl.BlockSpec((RB, F), lambda r: (r, 0)),
    )(s, hp, degp)


def kernel(x, edge_index, W1, W2, W3):
    ei = edge_index.astype(jnp.int32)
    src3 = ei[0].reshape(NS, NCH, C)
    dst3 = ei[1].reshape(NS, NCH, C)
    dst4 = ei[1].reshape(NC * NS, NCH_D, C)

    degp = _deg_kernel(dst4)

    w1p = jnp.pad(W1, ((0, 0), (0, F - F_IN)))
    wstack = jnp.stack([w1p, W2, W3]).reshape(3, NC, Q, W64, F)
    flags = jnp.array([1.0, 0.0, 0.0], jnp.float32).reshape(3, 1, 1)

    hp0 = _hp0(x, degp)
    s0 = jnp.zeros((NC, Q, N, W64), jnp.float32)

    def step(carry, xs):
        s_prev, hp_prev = carry
        wl, flag = xs
        hp = _mm(s_prev, hp_prev, degp, wl, flag)
        s = _prop_kernel(hp, src3, dst3)
        return (s, hp), None

    (s3, hp3), _ = lax.scan(step, (s0, hp0), (wstack, flags))
    return _final(s3, hp3, degp)


# speedup vs baseline: 1.0538x; 1.0538x over previous
import functools
import jax
import jax.numpy as jnp
from jax import lax
from jax.experimental import pallas as pl
from jax.experimental.pallas import tpu as pltpu
from jax.experimental.pallas import tpu_sc as plsc

N = 10000
E = 320000
F_IN = 128
F = 256
H = 128
Q = 2
W64 = H // Q
NC, NS = 2, 16
C = 80
NBUF = 5
EPT = E // NS
NCH = EPT // C
EPW = E // (NC * NS)
NCH_D = EPW // C
RPT = 624
RB = 1000

_mesh = plsc.VectorSubcoreMesh(core_axis_name="c", subcore_axis_name="s",
                               num_cores=NC, num_subcores=NS)


def _zero_vmem(buf, rows, width):
    z = jnp.zeros((16,), jnp.float32)

    def zr(r, _):
        def zc(k, _):
            buf[r, pl.ds(k * 16, 16)] = z
            return 0
        lax.fori_loop(0, width // 16, zc, 0)
        return 0
    lax.fori_loop(0, rows, zr, 0)


def _zero_stripe(acc, zbuf, base, rows, chunk):
    nfull = rows // chunk
    rem = rows % chunk
    for t in range(nfull):
        pltpu.sync_copy(zbuf, acc.at[pl.ds(base + t * chunk, chunk)])
    if rem:
        pltpu.sync_copy(zbuf.at[pl.ds(0, rem)],
                        acc.at[pl.ds(base + nfull * chunk, rem)])


@functools.partial(
    pl.kernel,
    out_type=jax.ShapeDtypeStruct((NC, N, 16), jnp.float32),
    mesh=_mesh,
    scratch_types=[
        pltpu.VMEM((NCH_D, C), jnp.int32),
        pltpu.VMEM((C, 16), jnp.float32),
        pltpu.VMEM((C, 16), jnp.float32),
        pltpu.VMEM_SHARED((N, 16), jnp.float32),
    ],
    compiler_params=pltpu.CompilerParams(use_tc_tiling_on_sc=False),
)
def _deg_kernel(dst_hbm, out_hbm, dstbuf, onesbuf, zbuf, acc):
    c = lax.axis_index("c")
    s = lax.axis_index("s")
    w = s * NC + c
    e0 = jnp.where(lax.iota(jnp.int32, 16) == 0, 1.0, 0.0).astype(jnp.float32)
    z = jnp.zeros((16,), jnp.float32)

    def fill(r, _):
        onesbuf[r] = e0
        zbuf[r] = z
        return 0
    lax.fori_loop(0, C, fill, 0)

    base = pl.multiple_of(s * RPT, 8)
    _zero_stripe(acc, zbuf, base, RPT, C)

    @pl.when(s == NS - 1)
    def _():
        _zero_stripe(acc, zbuf, N - 16, 16, C)

    plsc.subcore_barrier()

    pltpu.sync_copy(dst_hbm.at[w], dstbuf)

    def body(j, _):
        pltpu.sync_copy(onesbuf, acc.at[dstbuf.at[j]], add=True)
        return 0
    lax.fori_loop(0, NCH_D, body, 0)

    plsc.subcore_barrier()
    pltpu.sync_copy(acc.at[pl.ds(base, RPT)],
                    out_hbm.at[c].at[pl.ds(base, RPT)])

    @pl.when(s == NS - 1)
    def _():
        pltpu.sync_copy(acc.at[pl.ds(N - 16, 16)],
                        out_hbm.at[c].at[pl.ds(N - 16, 16)])


@functools.partial(
    pl.kernel,
    out_type=jax.ShapeDtypeStruct((NC, Q, N, W64), jnp.float32),
    mesh=_mesh,
    scratch_types=[
        pltpu.VMEM((NCH, C), jnp.int32),
        pltpu.VMEM((NCH, C), jnp.int32),
        [pltpu.VMEM((C, W64), jnp.float32) for _ in range(NBUF)],
        pltpu.VMEM((C, W64), jnp.float32),
        pltpu.VMEM_SHARED((N, W64), jnp.float32),
        [pltpu.SemaphoreType.DMA for _ in range(NBUF)],
        [pltpu.SemaphoreType.DMA for _ in range(NBUF)],
    ],
    compiler_params=pltpu.CompilerParams(use_tc_tiling_on_sc=False),
)
def _prop_kernel(h4_hbm, src_hbm, dst_hbm, out_hbm,
                 srcbuf, dstbuf, gbufs, zbuf, acc, gsems, ssems):
    c = lax.axis_index("c")
    s = lax.axis_index("s")

    _zero_vmem(zbuf, C, W64)
    base = pl.multiple_of(s * RPT, 8)

    pltpu.sync_copy(src_hbm.at[s], srcbuf)
    pltpu.sync_copy(dst_hbm.at[s], dstbuf)

    for q in range(Q):
        _zero_stripe(acc, zbuf, base, RPT, C)

        @pl.when(s == NS - 1)
        def _():
            _zero_stripe(acc, zbuf, N - 16, 16, C)

        plsc.subcore_barrier()

        tab = h4_hbm.at[c].at[q]

        for b in range(NBUF):
            pltpu.async_copy(tab.at[srcbuf.at[b]], gbufs[b], gsems[b])

        def body(i, _):
            sds = []
            for b in range(NBUF):
                j = i * NBUF + b
                pltpu.make_async_copy(tab.at[srcbuf.at[0]],
                                      gbufs[b], gsems[b]).wait()
                sds.append(pltpu.async_copy(gbufs[b], acc.at[dstbuf.at[j]],
                                            ssems[b], add=True))
            for b in range(NBUF):
                j = i * NBUF + b + NBUF
                sds[b].wait()

                @pl.when(j < NCH)
                def _():
                    pltpu.async_copy(tab.at[srcbuf.at[j]], gbufs[b], gsems[b])
            return 0
        lax.fori_loop(0, NCH // NBUF, body, 0)

        plsc.subcore_barrier()

        outq = out_hbm.at[c].at[q]
        pltpu.sync_copy(acc.at[pl.ds(base, RPT)], outq.at[pl.ds(base, RPT)])

        @pl.when(s == NS - 1)
        def _():
            pltpu.sync_copy(acc.at[pl.ds(N - 16, 16)],
                            outq.at[pl.ds(N - 16, 16)])

        plsc.subcore_barrier()


def _dis_from_degp(degp_blk):
    deg = jnp.sum(degp_blk, axis=(0, 2)) + 1.0
    return lax.rsqrt(deg)


def _hp0_body(x_ref, degp_ref, o_ref):
    c = pl.program_id(0)
    deg = jnp.sum(degp_ref[...], axis=(0, 2)) + 1.0
    rdis = jnp.sqrt(deg).reshape(RB, 1)
    v = jnp.where(c == 0, x_ref[...] * rdis, 0.0)
    for qi in range(Q):
        o_ref[0, qi] = v[:, qi * W64:(qi + 1) * W64]


def _mm_body(s_ref, hp_ref, degp_ref, w_ref, flag_ref, o_ref):
    dis = _dis_from_degp(degp_ref[...]).reshape(1, 1, RB, 1)
    v = s_ref[...] + hp_ref[...]
    is_first = flag_ref[0, 0] > 0.0
    act = dis * jnp.where(is_first, v, jax.nn.relu(v))
    w = w_ref[0, 0]
    h = jnp.zeros((RB, W64), jnp.float32)
    for ci in range(NC):
        for qi in range(Q):
            wslice = w[:, ci * H + qi * W64: ci * H + (qi + 1) * W64]
            h = h + lax.dot_general(act[ci, qi], wslice,
                                    (((1,), (1,)), ((), ())),
                                    preferred_element_type=jnp.float32)
    o_ref[...] = (dis[0, 0] * h)[None, None]


def _final_body(s_ref, hp_ref, degp_ref, o_ref):
    dis = _dis_from_degp(degp_ref[...]).reshape(1, 1, RB, 1)
    v = dis * (s_ref[...] + hp_ref[...])
    for ci in range(NC):
        for qi in range(Q):
            lo = ci * H + qi * W64
            o_ref[:, lo:lo + W64] = v[ci, qi]


def _hp0(x, degp):
    return pl.pallas_call(
        _hp0_body,
        out_shape=jax.ShapeDtypeStruct((NC, Q, N, W64), jnp.float32),
        grid=(NC, N // RB),
        in_specs=[
            pl.BlockSpec((RB, F_IN), lambda c, r: (r, 0)),
            pl.BlockSpec((NC, RB, 16), lambda c, r: (0, r, 0)),
        ],
        out_specs=pl.BlockSpec((1, Q, RB, W64), lambda c, r: (c, 0, r, 0)),
    )(x, degp)


def _mm(s, hp, degp, wr, flag):
    return pl.pallas_call(
        _mm_body,
        out_shape=jax.ShapeDtypeStruct((NC, Q, N, W64), jnp.float32),
        grid=(N // RB, NC, Q),
        in_specs=[
            pl.BlockSpec((NC, Q, RB, W64), lambda r, c, q: (0, 0, r, 0)),
            pl.BlockSpec((NC, Q, RB, W64), lambda r, c, q: (0, 0, r, 0)),
            pl.BlockSpec((NC, RB, 16), lambda r, c, q: (0, r, 0)),
            pl.BlockSpec((1, 1, W64, F), lambda r, c, q: (c, q, 0, 0)),
            pl.BlockSpec((1, 1), lambda r, c, q: (0, 0)),
        ],
        out_specs=pl.BlockSpec((1, 1, RB, W64), lambda r, c, q: (c, q, r, 0)),
    )(s, hp, degp, wr, flag)


def _final(s, hp, degp):
    return pl.pallas_call(
        _final_body,
        out_shape=jax.ShapeDtypeStruct((N, F), jnp.float32),
        grid=(N // RB,),
        in_specs=[
            pl.BlockSpec((NC, Q, RB, W64), lambda r: (0, 0, r, 0)),
            pl.BlockSpec((NC, Q, RB, W64), lambda r: (0, 0, r, 0)),
            pl.BlockSpec((NC, RB, 16), lambda r: (0, r, 0)),
        ],
        out_specs=pl.BlockSpec((RB, F), lambda r: (r, 0)),
    )(s, hp, degp)


def kernel(x, edge_index, W1, W2, W3):
    ei = edge_index.astype(jnp.int32)
    src3 = ei[0].reshape(NS, NCH, C)
    dst3 = ei[1].reshape(NS, NCH, C)
    dst4 = ei[1].reshape(NC * NS, NCH_D, C)

    degp = _deg_kernel(dst4)

    w1p = jnp.pad(W1, ((0, 0), (0, F - F_IN)))
    wstack = jnp.stack([w1p, W2, W3]).reshape(3, NC, Q, W64, F)
    flags = jnp.array([1.0, 0.0, 0.0], jnp.float32).reshape(3, 1, 1)

    hp0 = _hp0(x, degp)
    s0 = jnp.zeros((NC, Q, N, W64), jnp.float32)

    def step(carry, xs):
        s_prev, hp_prev = carry
        wl, flag = xs
        hp = _mm(s_prev, hp_prev, degp, wl, flag)
        s = _prop_kernel(hp, src3, dst3)
        return (s, hp), None

    (s3, hp3), _ = lax.scan(step, (s0, hp0), (wstack, flags))
    return _final(s3, hp3, degp)


# speedup vs baseline: 1.2341x; 1.1710x over previous
import functools
import jax
import jax.numpy as jnp
from jax import lax
from jax.experimental import pallas as pl
from jax.experimental.pallas import tpu as pltpu
from jax.experimental.pallas import tpu_sc as plsc

N = 10000
E = 320000
F_IN = 128
F = 256
H = 128
Q = 2
W64 = H // Q
NC, NS = 2, 16
C = 80
NBUF = 5
EPT = E // NS
NCH = EPT // C
EPW = E // (NC * NS)
NCH_D = EPW // C
RPT = 624
RB = 1000

_mesh = plsc.VectorSubcoreMesh(core_axis_name="c", subcore_axis_name="s",
                               num_cores=NC, num_subcores=NS)


def _zero_vmem(buf, rows, width):
    z = jnp.zeros((16,), jnp.float32)

    def zr(r, _):
        def zc(k, _):
            buf[r, pl.ds(k * 16, 16)] = z
            return 0
        lax.fori_loop(0, width // 16, zc, 0)
        return 0
    lax.fori_loop(0, rows, zr, 0)


def _zero_stripe(acc, zbuf, base, rows, chunk):
    nfull = rows // chunk
    rem = rows % chunk
    for t in range(nfull):
        pltpu.sync_copy(zbuf, acc.at[pl.ds(base + t * chunk, chunk)])
    if rem:
        pltpu.sync_copy(zbuf.at[pl.ds(0, rem)],
                        acc.at[pl.ds(base + nfull * chunk, rem)])


@functools.partial(
    pl.kernel,
    out_type=jax.ShapeDtypeStruct((NC, N, 16), jnp.float32),
    mesh=_mesh,
    scratch_types=[
        pltpu.VMEM((NCH_D, C), jnp.int32),
        pltpu.VMEM((C, 16), jnp.float32),
        pltpu.VMEM((C, 16), jnp.float32),
        pltpu.VMEM_SHARED((N, 16), jnp.float32),
    ],
    compiler_params=pltpu.CompilerParams(use_tc_tiling_on_sc=False),
)
def _deg_kernel(dst_hbm, out_hbm, dstbuf, onesbuf, zbuf, acc):
    c = lax.axis_index("c")
    s = lax.axis_index("s")
    w = s * NC + c
    e0 = jnp.where(lax.iota(jnp.int32, 16) == 0, 1.0, 0.0).astype(jnp.float32)
    z = jnp.zeros((16,), jnp.float32)

    def fill(r, _):
        onesbuf[r] = e0
        zbuf[r] = z
        return 0
    lax.fori_loop(0, C, fill, 0)

    base = pl.multiple_of(s * RPT, 8)
    _zero_stripe(acc, zbuf, base, RPT, C)

    @pl.when(s == NS - 1)
    def _():
        _zero_stripe(acc, zbuf, N - 16, 16, C)

    plsc.subcore_barrier()

    pltpu.sync_copy(dst_hbm.at[w], dstbuf)

    def body(j, _):
        pltpu.sync_copy(onesbuf, acc.at[dstbuf.at[j]], add=True)
        return 0
    lax.fori_loop(0, NCH_D, body, 0)

    plsc.subcore_barrier()
    pltpu.sync_copy(acc.at[pl.ds(base, RPT)],
                    out_hbm.at[c].at[pl.ds(base, RPT)])

    @pl.when(s == NS - 1)
    def _():
        pltpu.sync_copy(acc.at[pl.ds(N - 16, 16)],
                        out_hbm.at[c].at[pl.ds(N - 16, 16)])


@functools.partial(
    pl.kernel,
    out_type=jax.ShapeDtypeStruct((N, F), jnp.float32),
    mesh=_mesh,
    scratch_types=[
        pltpu.VMEM((NCH, C), jnp.int32),
        pltpu.VMEM((NCH, C), jnp.int32),
        [pltpu.VMEM((C, W64), jnp.float32) for _ in range(NBUF)],
        pltpu.VMEM((C, W64), jnp.float32),
        pltpu.VMEM_SHARED((N, W64), jnp.float32),
        [pltpu.SemaphoreType.DMA for _ in range(NBUF)],
        [pltpu.SemaphoreType.DMA for _ in range(NBUF)],
    ],
    compiler_params=pltpu.CompilerParams(use_tc_tiling_on_sc=False),
)
def _prop_kernel(h4_hbm, src_hbm, dst_hbm, out_hbm,
                 srcbuf, dstbuf, gbufs, zbuf, acc, gsems, ssems):
    c = lax.axis_index("c")
    s = lax.axis_index("s")

    _zero_vmem(zbuf, C, W64)
    base = pl.multiple_of(s * RPT, 8)

    pltpu.sync_copy(src_hbm.at[s], srcbuf)
    pltpu.sync_copy(dst_hbm.at[s], dstbuf)

    for q in range(Q):
        _zero_stripe(acc, zbuf, base, RPT, C)

        @pl.when(s == NS - 1)
        def _():
            _zero_stripe(acc, zbuf, N - 16, 16, C)

        plsc.subcore_barrier()

        tab = h4_hbm.at[c].at[q]

        for b in range(NBUF):
            pltpu.async_copy(tab.at[srcbuf.at[b]], gbufs[b], gsems[b])

        def body(i, _):
            sds = []
            for b in range(NBUF):
                j = i * NBUF + b
                pltpu.make_async_copy(tab.at[srcbuf.at[0]],
                                      gbufs[b], gsems[b]).wait()
                sds.append(pltpu.async_copy(gbufs[b], acc.at[dstbuf.at[j]],
                                            ssems[b], add=True))
            for b in range(NBUF):
                j = i * NBUF + b + NBUF
                sds[b].wait()

                @pl.when(j < NCH)
                def _():
                    pltpu.async_copy(tab.at[srcbuf.at[j]], gbufs[b], gsems[b])
            return 0
        lax.fori_loop(0, NCH // NBUF, body, 0)

        plsc.subcore_barrier()

        outq = out_hbm.at[:, pl.ds(c * H + q * W64, W64)]
        pltpu.sync_copy(acc.at[pl.ds(base, RPT)], outq.at[pl.ds(base, RPT)])

        @pl.when(s == NS - 1)
        def _():
            pltpu.sync_copy(acc.at[pl.ds(N - 16, 16)],
                            outq.at[pl.ds(N - 16, 16)])

        plsc.subcore_barrier()


def _dis_from_degp(degp_blk):
    deg = jnp.sum(degp_blk, axis=(0, 2)) + 1.0
    return lax.rsqrt(deg)


def _hp0_body(x_ref, degp_ref, o_ref):
    deg = jnp.sum(degp_ref[...], axis=(0, 2)) + 1.0
    rdis = jnp.sqrt(deg).reshape(RB, 1)
    o_ref[:, :F_IN] = x_ref[...] * rdis
    o_ref[:, F_IN:] = jnp.zeros((RB, F - F_IN), jnp.float32)


def _mm_body(s_ref, hp_ref, degp_ref, w_ref, flag_ref, o_ref, oq_ref):
    dis = _dis_from_degp(degp_ref[...]).reshape(RB, 1)
    v = s_ref[...] + hp_ref[...]
    is_first = flag_ref[0, 0] > 0.0
    act = dis * jnp.where(is_first, v, jax.nn.relu(v))
    h = lax.dot_general(act, w_ref[...], (((1,), (1,)), ((), ())),
                        preferred_element_type=jnp.float32)
    hp = dis * h
    o_ref[...] = hp
    for ci in range(NC):
        for qi in range(Q):
            lo = ci * H + qi * W64
            oq_ref[ci, qi] = hp[:, lo:lo + W64]


def _final_body(s_ref, hp_ref, degp_ref, o_ref):
    dis = _dis_from_degp(degp_ref[...]).reshape(RB, 1)
    o_ref[...] = dis * (s_ref[...] + hp_ref[...])


def _hp0(x, degp):
    return pl.pallas_call(
        _hp0_body,
        out_shape=jax.ShapeDtypeStruct((N, F), jnp.float32),
        grid=(N // RB,),
        in_specs=[
            pl.BlockSpec((RB, F_IN), lambda r: (r, 0)),
            pl.BlockSpec((NC, RB, 16), lambda r: (0, r, 0)),
        ],
        out_specs=pl.BlockSpec((RB, F), lambda r: (r, 0)),
    )(x, degp)


def _mm(s, hp, degp, wr, flag):
    return pl.pallas_call(
        _mm_body,
        out_shape=[jax.ShapeDtypeStruct((N, F), jnp.float32),
                   jax.ShapeDtypeStruct((NC, Q, N, W64), jnp.float32)],
        grid=(N // RB,),
        in_specs=[
            pl.BlockSpec((RB, F), lambda r: (r, 0)),
            pl.BlockSpec((RB, F), lambda r: (r, 0)),
            pl.BlockSpec((NC, RB, 16), lambda r: (0, r, 0)),
            pl.BlockSpec((F, F), lambda r: (0, 0)),
            pl.BlockSpec((1, 1), lambda r: (0, 0)),
        ],
        out_specs=[pl.BlockSpec((RB, F), lambda r: (r, 0)),
                   pl.BlockSpec((NC, Q, RB, W64), lambda r: (0, 0, r, 0))],
    )(s, hp, degp, wr, flag)


def _final(s, hp, degp):
    return pl.pallas_call(
        _final_body,
        out_shape=jax.ShapeDtypeStruct((N, F), jnp.float32),
        grid=(N // RB,),
        in_specs=[
            pl.BlockSpec((RB, F), lambda r: (r, 0)),
            pl.BlockSpec((RB, F), lambda r: (r, 0)),
            pl.BlockSpec((NC, RB, 16), lambda r: (0, r, 0)),
        ],
        out_specs=pl.BlockSpec((RB, F), lambda r: (r, 0)),
    )(s, hp, degp)


def kernel(x, edge_index, W1, W2, W3):
    ei = edge_index.astype(jnp.int32)
    src3 = ei[0].reshape(NS, NCH, C)
    dst3 = ei[1].reshape(NS, NCH, C)
    dst4 = ei[1].reshape(NC * NS, NCH_D, C)

    degp = _deg_kernel(dst4)

    w1p = jnp.pad(W1, ((0, 0), (0, F - F_IN)))
    wstack = jnp.stack([w1p, W2, W3])
    flags = jnp.array([1.0, 0.0, 0.0], jnp.float32).reshape(3, 1, 1)

    hp0 = _hp0(x, degp)
    s0 = jnp.zeros((N, F), jnp.float32)

    def step(carry, xs):
        s_prev, hp_prev = carry
        wl, flag = xs
        hp, hp_q = _mm(s_prev, hp_prev, degp, wl, flag)
        s = _prop_kernel(hp_q, src3, dst3)
        return (s, hp), None

    (s3, hp3), _ = lax.scan(step, (s0, hp0), (wstack, flags))
    return _final(s3, hp3, degp)


# speedup vs baseline: 1.2382x; 1.0034x over previous
import functools
import jax
import jax.numpy as jnp
from jax import lax
from jax.experimental import pallas as pl
from jax.experimental.pallas import tpu as pltpu
from jax.experimental.pallas import tpu_sc as plsc

N = 10000
E = 320000
F_IN = 128
F = 256
H = 128
Q = 2
W64 = H // Q
NC, NS = 2, 16
C = 80
NBUF = 5
EPT = E // NS
NCH = EPT // C
EPW = E // (NC * NS)
NCH_D = EPW // C
RPT = 624
RB = 1000

_mesh = plsc.VectorSubcoreMesh(core_axis_name="c", subcore_axis_name="s",
                               num_cores=NC, num_subcores=NS)


def _zero_vmem(buf, rows, width):
    z = jnp.zeros((16,), jnp.float32)

    def zr(r, _):
        def zc(k, _):
            buf[r, pl.ds(k * 16, 16)] = z
            return 0
        lax.fori_loop(0, width // 16, zc, 0)
        return 0
    lax.fori_loop(0, rows, zr, 0)


def _zero_stripe(acc, zbuf, base, rows, chunk, sem):
    nfull = rows // chunk
    rem = rows % chunk
    ds = []
    for t in range(nfull):
        ds.append(pltpu.async_copy(
            zbuf, acc.at[pl.ds(base + t * chunk, chunk)], sem))
    if rem:
        ds.append(pltpu.async_copy(
            zbuf.at[pl.ds(0, rem)],
            acc.at[pl.ds(base + nfull * chunk, rem)], sem))
    for d in ds:
        d.wait()


@functools.partial(
    pl.kernel,
    out_type=jax.ShapeDtypeStruct((NC, N, 16), jnp.float32),
    mesh=_mesh,
    scratch_types=[
        pltpu.VMEM((NCH_D, C), jnp.int32),
        pltpu.VMEM((C, 16), jnp.float32),
        pltpu.VMEM((C, 16), jnp.float32),
        pltpu.VMEM_SHARED((N, 16), jnp.float32),
        pltpu.SemaphoreType.DMA,
    ],
    compiler_params=pltpu.CompilerParams(use_tc_tiling_on_sc=False),
)
def _deg_kernel(dst_hbm, out_hbm, dstbuf, onesbuf, zbuf, acc, zsem):
    c = lax.axis_index("c")
    s = lax.axis_index("s")
    w = s * NC + c
    e0 = jnp.where(lax.iota(jnp.int32, 16) == 0, 1.0, 0.0).astype(jnp.float32)
    z = jnp.zeros((16,), jnp.float32)

    def fill(r, _):
        onesbuf[r] = e0
        zbuf[r] = z
        return 0
    lax.fori_loop(0, C, fill, 0)

    base = pl.multiple_of(s * RPT, 8)
    _zero_stripe(acc, zbuf, base, RPT, C, zsem)

    @pl.when(s == NS - 1)
    def _():
        _zero_stripe(acc, zbuf, N - 16, 16, C, zsem)

    plsc.subcore_barrier()

    pltpu.sync_copy(dst_hbm.at[w], dstbuf)

    def body(j, _):
        pltpu.sync_copy(onesbuf, acc.at[dstbuf.at[j]], add=True)
        return 0
    lax.fori_loop(0, NCH_D, body, 0)

    plsc.subcore_barrier()
    pltpu.sync_copy(acc.at[pl.ds(base, RPT)],
                    out_hbm.at[c].at[pl.ds(base, RPT)])

    @pl.when(s == NS - 1)
    def _():
        pltpu.sync_copy(acc.at[pl.ds(N - 16, 16)],
                        out_hbm.at[c].at[pl.ds(N - 16, 16)])


@functools.partial(
    pl.kernel,
    out_type=jax.ShapeDtypeStruct((N, F), jnp.float32),
    mesh=_mesh,
    scratch_types=[
        pltpu.VMEM((NCH, C), jnp.int32),
        pltpu.VMEM((NCH, C), jnp.int32),
        [pltpu.VMEM((C, W64), jnp.float32) for _ in range(NBUF)],
        pltpu.VMEM((C, W64), jnp.float32),
        pltpu.VMEM_SHARED((N, W64), jnp.float32),
        [pltpu.SemaphoreType.DMA for _ in range(NBUF)],
        [pltpu.SemaphoreType.DMA for _ in range(NBUF)],
        pltpu.SemaphoreType.DMA,
    ],
    compiler_params=pltpu.CompilerParams(use_tc_tiling_on_sc=False),
)
def _prop_kernel(h4_hbm, src_hbm, dst_hbm, out_hbm,
                 srcbuf, dstbuf, gbufs, zbuf, acc, gsems, ssems, zsem):
    c = lax.axis_index("c")
    s = lax.axis_index("s")

    _zero_vmem(zbuf, C, W64)
    base = pl.multiple_of(s * RPT, 8)

    pltpu.sync_copy(src_hbm.at[s], srcbuf)
    pltpu.sync_copy(dst_hbm.at[s], dstbuf)

    for q in range(Q):
        _zero_stripe(acc, zbuf, base, RPT, C, zsem)

        @pl.when(s == NS - 1)
        def _():
            _zero_stripe(acc, zbuf, N - 16, 16, C, zsem)

        plsc.subcore_barrier()

        tab = h4_hbm.at[c].at[q]

        for b in range(NBUF):
            pltpu.async_copy(tab.at[srcbuf.at[b]], gbufs[b], gsems[b])

        def body(i, _):
            sds = []
            for b in range(NBUF):
                j = i * NBUF + b
                pltpu.make_async_copy(tab.at[srcbuf.at[0]],
                                      gbufs[b], gsems[b]).wait()
                sds.append(pltpu.async_copy(gbufs[b], acc.at[dstbuf.at[j]],
                                            ssems[b], add=True))
            for b in range(NBUF):
                j = i * NBUF + b + NBUF
                sds[b].wait()

                @pl.when(j < NCH)
                def _():
                    pltpu.async_copy(tab.at[srcbuf.at[j]], gbufs[b], gsems[b])
            return 0
        lax.fori_loop(0, NCH // NBUF, body, 0)

        plsc.subcore_barrier()

        outq = out_hbm.at[:, pl.ds(c * H + q * W64, W64)]
        pltpu.sync_copy(acc.at[pl.ds(base, RPT)], outq.at[pl.ds(base, RPT)])

        @pl.when(s == NS - 1)
        def _():
            pltpu.sync_copy(acc.at[pl.ds(N - 16, 16)],
                            outq.at[pl.ds(N - 16, 16)])

        plsc.subcore_barrier()


def _dis_from_degp(degp_blk):
    deg = jnp.sum(degp_blk, axis=(0, 2)) + 1.0
    return lax.rsqrt(deg)


def _hp0_body(x_ref, degp_ref, o_ref):
    deg = jnp.sum(degp_ref[...], axis=(0, 2)) + 1.0
    rdis = jnp.sqrt(deg).reshape(RB, 1)
    o_ref[:, :F_IN] = x_ref[...] * rdis
    o_ref[:, F_IN:] = jnp.zeros((RB, F - F_IN), jnp.float32)


def _mm_body(s_ref, hp_ref, degp_ref, w_ref, flag_ref, o_ref, oq_ref):
    dis = _dis_from_degp(degp_ref[...]).reshape(RB, 1)
    v = s_ref[...] + hp_ref[...]
    is_first = flag_ref[0, 0] > 0.0
    act = dis * jnp.where(is_first, v, jax.nn.relu(v))
    h = lax.dot_general(act, w_ref[...], (((1,), (1,)), ((), ())),
                        preferred_element_type=jnp.float32)
    hp = dis * h
    o_ref[...] = hp
    for ci in range(NC):
        for qi in range(Q):
            lo = ci * H + qi * W64
            oq_ref[ci, qi] = hp[:, lo:lo + W64]


def _final_body(s_ref, hp_ref, degp_ref, o_ref):
    dis = _dis_from_degp(degp_ref[...]).reshape(RB, 1)
    o_ref[...] = dis * (s_ref[...] + hp_ref[...])


def _hp0(x, degp):
    return pl.pallas_call(
        _hp0_body,
        out_shape=jax.ShapeDtypeStruct((N, F), jnp.float32),
        grid=(N // RB,),
        in_specs=[
            pl.BlockSpec((RB, F_IN), lambda r: (r, 0)),
            pl.BlockSpec((NC, RB, 16), lambda r: (0, r, 0)),
        ],
        out_specs=pl.BlockSpec((RB, F), lambda r: (r, 0)),
    )(x, degp)


def _mm(s, hp, degp, wr, flag):
    return pl.pallas_call(
        _mm_body,
        out_shape=[jax.ShapeDtypeStruct((N, F), jnp.float32),
                   jax.ShapeDtypeStruct((NC, Q, N, W64), jnp.float32)],
        grid=(N // RB,),
        in_specs=[
            pl.BlockSpec((RB, F), lambda r: (r, 0)),
            pl.BlockSpec((RB, F), lambda r: (r, 0)),
            pl.BlockSpec((NC, RB, 16), lambda r: (0, r, 0)),
            pl.BlockSpec((F, F), lambda r: (0, 0)),
            pl.BlockSpec((1, 1), lambda r: (0, 0)),
        ],
        out_specs=[pl.BlockSpec((RB, F), lambda r: (r, 0)),
                   pl.BlockSpec((NC, Q, RB, W64), lambda r: (0, 0, r, 0))],
    )(s, hp, degp, wr, flag)


def _final(s, hp, degp):
    return pl.pallas_call(
        _final_body,
        out_shape=jax.ShapeDtypeStruct((N, F), jnp.float32),
        grid=(N // RB,),
        in_specs=[
            pl.BlockSpec((RB, F), lambda r: (r, 0)),
            pl.BlockSpec((RB, F), lambda r: (r, 0)),
            pl.BlockSpec((NC, RB, 16), lambda r: (0, r, 0)),
        ],
        out_specs=pl.BlockSpec((RB, F), lambda r: (r, 0)),
    )(s, hp, degp)


def kernel(x, edge_index, W1, W2, W3):
    ei = edge_index.astype(jnp.int32)
    src3 = ei[0].reshape(NS, NCH, C)
    dst3 = ei[1].reshape(NS, NCH, C)
    dst4 = ei[1].reshape(NC * NS, NCH_D, C)

    degp = _deg_kernel(dst4)

    w1p = jnp.pad(W1, ((0, 0), (0, F - F_IN)))
    wstack = jnp.stack([w1p, W2, W3])
    flags = jnp.array([1.0, 0.0, 0.0], jnp.float32).reshape(3, 1, 1)

    hp0 = _hp0(x, degp)
    s0 = jnp.zeros((N, F), jnp.float32)

    def step(carry, xs):
        s_prev, hp_prev = carry
        wl, flag = xs
        hp, hp_q = _mm(s_prev, hp_prev, degp, wl, flag)
        s = _prop_kernel(hp_q, src3, dst3)
        return (s, hp), None

    (s3, hp3), _ = lax.scan(step, (s0, hp0), (wstack, flags))
    return _final(s3, hp3, degp)


# speedup vs baseline: 1.2485x; 1.0083x over previous
import functools
import jax
import jax.numpy as jnp
from jax import lax
from jax.experimental import pallas as pl
from jax.experimental.pallas import tpu as pltpu
from jax.experimental.pallas import tpu_sc as plsc

N = 10000
E = 320000
F_IN = 128
F = 256
H = 128
Q = 2
W64 = H // Q
NC, NS = 2, 16
C = 80
NBUF = 5
EPT = E // NS
NCH = EPT // C
EPW = E // (NC * NS)
NCH_D = EPW // C
RPT = 624
RB = 1000

_mesh = plsc.VectorSubcoreMesh(core_axis_name="c", subcore_axis_name="s",
                               num_cores=NC, num_subcores=NS)


def _zero_vmem(buf, rows, width):
    z = jnp.zeros((16,), jnp.float32)

    def zr(r, _):
        def zc(k, _):
            buf[r, pl.ds(k * 16, 16)] = z
            return 0
        lax.fori_loop(0, width // 16, zc, 0)
        return 0
    lax.fori_loop(0, rows, zr, 0)


def _zero_stripe(acc, zbuf, base, rows, chunk, sem):
    nfull = rows // chunk
    rem = rows % chunk
    ds = []
    for t in range(nfull):
        ds.append(pltpu.async_copy(
            zbuf, acc.at[pl.ds(base + t * chunk, chunk)], sem))
    if rem:
        ds.append(pltpu.async_copy(
            zbuf.at[pl.ds(0, rem)],
            acc.at[pl.ds(base + nfull * chunk, rem)], sem))
    for d in ds:
        d.wait()


@functools.partial(
    pl.kernel,
    out_type=jax.ShapeDtypeStruct((NC, N, 16), jnp.float32),
    mesh=_mesh,
    scratch_types=[
        pltpu.VMEM((NCH_D, C), jnp.int32),
        pltpu.VMEM((C, 16), jnp.float32),
        pltpu.VMEM((C, 16), jnp.float32),
        pltpu.VMEM_SHARED((N, 16), jnp.float32),
        pltpu.SemaphoreType.DMA,
    ],
    compiler_params=pltpu.CompilerParams(use_tc_tiling_on_sc=False),
)
def _deg_kernel(dst_hbm, out_hbm, dstbuf, onesbuf, zbuf, acc, zsem):
    c = lax.axis_index("c")
    s = lax.axis_index("s")
    w = s * NC + c
    e0 = jnp.where(lax.iota(jnp.int32, 16) == 0, 1.0, 0.0).astype(jnp.float32)
    z = jnp.zeros((16,), jnp.float32)

    def fill(r, _):
        onesbuf[r] = e0
        zbuf[r] = z
        return 0
    lax.fori_loop(0, C, fill, 0)

    base = pl.multiple_of(s * RPT, 8)
    _zero_stripe(acc, zbuf, base, RPT, C, zsem)

    @pl.when(s == NS - 1)
    def _():
        _zero_stripe(acc, zbuf, N - 16, 16, C, zsem)

    plsc.subcore_barrier()

    pltpu.sync_copy(dst_hbm.at[w], dstbuf)

    def body(j, _):
        pltpu.async_copy(onesbuf, acc.at[dstbuf.at[j]], zsem, add=True)
        return 0
    lax.fori_loop(0, NCH_D, body, 0)

    def drain(j, _):
        pltpu.make_async_copy(onesbuf, acc.at[dstbuf.at[0]], zsem).wait()
        return 0
    lax.fori_loop(0, NCH_D, drain, 0)

    plsc.subcore_barrier()
    pltpu.sync_copy(acc.at[pl.ds(base, RPT)],
                    out_hbm.at[c].at[pl.ds(base, RPT)])

    @pl.when(s == NS - 1)
    def _():
        pltpu.sync_copy(acc.at[pl.ds(N - 16, 16)],
                        out_hbm.at[c].at[pl.ds(N - 16, 16)])


@functools.partial(
    pl.kernel,
    out_type=jax.ShapeDtypeStruct((N, F), jnp.float32),
    mesh=_mesh,
    scratch_types=[
        pltpu.VMEM((NCH, C), jnp.int32),
        pltpu.VMEM((NCH, C), jnp.int32),
        [pltpu.VMEM((C, W64), jnp.float32) for _ in range(NBUF)],
        pltpu.VMEM((C, W64), jnp.float32),
        pltpu.VMEM_SHARED((N, W64), jnp.float32),
        [pltpu.SemaphoreType.DMA for _ in range(NBUF)],
        [pltpu.SemaphoreType.DMA for _ in range(NBUF)],
        pltpu.SemaphoreType.DMA,
    ],
    compiler_params=pltpu.CompilerParams(use_tc_tiling_on_sc=False),
)
def _prop_kernel(h4_hbm, src_hbm, dst_hbm, out_hbm,
                 srcbuf, dstbuf, gbufs, zbuf, acc, gsems, ssems, zsem):
    c = lax.axis_index("c")
    s = lax.axis_index("s")

    _zero_vmem(zbuf, C, W64)
    base = pl.multiple_of(s * RPT, 8)

    pltpu.sync_copy(src_hbm.at[s], srcbuf)
    pltpu.sync_copy(dst_hbm.at[s], dstbuf)

    for q in range(Q):
        _zero_stripe(acc, zbuf, base, RPT, C, zsem)

        @pl.when(s == NS - 1)
        def _():
            _zero_stripe(acc, zbuf, N - 16, 16, C, zsem)

        plsc.subcore_barrier()

        tab = h4_hbm.at[c].at[q]

        for b in range(NBUF):
            pltpu.async_copy(tab.at[srcbuf.at[b]], gbufs[b], gsems[b])

        def body(i, _):
            sds = []
            for b in range(NBUF):
                j = i * NBUF + b
                pltpu.make_async_copy(tab.at[srcbuf.at[0]],
                                      gbufs[b], gsems[b]).wait()
                sds.append(pltpu.async_copy(gbufs[b], acc.at[dstbuf.at[j]],
                                            ssems[b], add=True))
            for b in range(NBUF):
                j = i * NBUF + b + NBUF
                sds[b].wait()

                @pl.when(j < NCH)
                def _():
                    pltpu.async_copy(tab.at[srcbuf.at[j]], gbufs[b], gsems[b])
            return 0
        lax.fori_loop(0, NCH // NBUF, body, 0)

        plsc.subcore_barrier()

        outq = out_hbm.at[:, pl.ds(c * H + q * W64, W64)]
        pltpu.sync_copy(acc.at[pl.ds(base, RPT)], outq.at[pl.ds(base, RPT)])

        @pl.when(s == NS - 1)
        def _():
            pltpu.sync_copy(acc.at[pl.ds(N - 16, 16)],
                            outq.at[pl.ds(N - 16, 16)])

        plsc.subcore_barrier()


def _dis_from_degp(degp_blk):
    deg = jnp.sum(degp_blk, axis=(0, 2)) + 1.0
    return lax.rsqrt(deg)


def _hp0_body(x_ref, degp_ref, o_ref):
    deg = jnp.sum(degp_ref[...], axis=(0, 2)) + 1.0
    rdis = jnp.sqrt(deg).reshape(RB, 1)
    o_ref[:, :F_IN] = x_ref[...] * rdis
    o_ref[:, F_IN:] = jnp.zeros((RB, F - F_IN), jnp.float32)


def _mm_body(s_ref, hp_ref, degp_ref, w_ref, flag_ref, o_ref, oq_ref):
    dis = _dis_from_degp(degp_ref[...]).reshape(RB, 1)
    v = s_ref[...] + hp_ref[...]
    is_first = flag_ref[0, 0] > 0.0
    act = dis * jnp.where(is_first, v, jax.nn.relu(v))
    h = lax.dot_general(act, w_ref[...], (((1,), (1,)), ((), ())),
                        preferred_element_type=jnp.float32)
    hp = dis * h
    o_ref[...] = hp
    for ci in range(NC):
        for qi in range(Q):
            lo = ci * H + qi * W64
            oq_ref[ci, qi] = hp[:, lo:lo + W64]


def _final_body(s_ref, hp_ref, degp_ref, o_ref):
    dis = _dis_from_degp(degp_ref[...]).reshape(RB, 1)
    o_ref[...] = dis * (s_ref[...] + hp_ref[...])


def _hp0(x, degp):
    return pl.pallas_call(
        _hp0_body,
        out_shape=jax.ShapeDtypeStruct((N, F), jnp.float32),
        grid=(N // RB,),
        in_specs=[
            pl.BlockSpec((RB, F_IN), lambda r: (r, 0)),
            pl.BlockSpec((NC, RB, 16), lambda r: (0, r, 0)),
        ],
        out_specs=pl.BlockSpec((RB, F), lambda r: (r, 0)),
    )(x, degp)


def _mm(s, hp, degp, wr, flag):
    return pl.pallas_call(
        _mm_body,
        out_shape=[jax.ShapeDtypeStruct((N, F), jnp.float32),
                   jax.ShapeDtypeStruct((NC, Q, N, W64), jnp.float32)],
        grid=(N // RB,),
        in_specs=[
            pl.BlockSpec((RB, F), lambda r: (r, 0)),
            pl.BlockSpec((RB, F), lambda r: (r, 0)),
            pl.BlockSpec((NC, RB, 16), lambda r: (0, r, 0)),
            pl.BlockSpec((F, F), lambda r: (0, 0)),
            pl.BlockSpec((1, 1), lambda r: (0, 0)),
        ],
        out_specs=[pl.BlockSpec((RB, F), lambda r: (r, 0)),
                   pl.BlockSpec((NC, Q, RB, W64), lambda r: (0, 0, r, 0))],
    )(s, hp, degp, wr, flag)


def _final(s, hp, degp):
    return pl.pallas_call(
        _final_body,
        out_shape=jax.ShapeDtypeStruct((N, F), jnp.float32),
        grid=(N // RB,),
        in_specs=[
            pl.BlockSpec((RB, F), lambda r: (r, 0)),
            pl.BlockSpec((RB, F), lambda r: (r, 0)),
            pl.BlockSpec((NC, RB, 16), lambda r: (0, r, 0)),
        ],
        out_specs=pl.BlockSpec((RB, F), lambda r: (r, 0)),
    )(s, hp, degp)


def kernel(x, edge_index, W1, W2, W3):
    ei = edge_index.astype(jnp.int32)
    src3 = ei[0].reshape(NS, NCH, C)
    dst3 = ei[1].reshape(NS, NCH, C)
    dst4 = ei[1].reshape(NC * NS, NCH_D, C)

    degp = _deg_kernel(dst4)

    w1p = jnp.pad(W1, ((0, 0), (0, F - F_IN)))
    wstack = jnp.stack([w1p, W2, W3])
    flags = jnp.array([1.0, 0.0, 0.0], jnp.float32).reshape(3, 1, 1)

    hp0 = _hp0(x, degp)
    s0 = jnp.zeros((N, F), jnp.float32)

    def step(carry, xs):
        s_prev, hp_prev = carry
        wl, flag = xs
        hp, hp_q = _mm(s_prev, hp_prev, degp, wl, flag)
        s = _prop_kernel(hp_q, src3, dst3)
        return (s, hp), None

    (s3, hp3), _ = lax.scan(step, (s0, hp0), (wstack, flags))
    return _final(s3, hp3, degp)
